# baseline (speedup 1.0000x reference)
import functools

import jax
import jax.numpy as jnp
from jax import lax
from jax.experimental import pallas as pl
from jax.experimental.pallas import tpu as pltpu
from jax.experimental.pallas import tpu_sc as plsc

_N_TABLES = 26
_VOCAB = 100000
_D = 32
_B = 16384
_NC = 2
_NS = 16
_NW = _NC * _NS
_NCH = 8
_CH = 12544
_CHB = 12160
_TAIL = _VOCAB - 7 * _CH - _CHB
_SZ = [_CH] * 7 + [_CHB]


def _body(pk_hbm, bnd_hbm, tab_hbm, tail_hbm, out_hbm,
          pk_v, slot_a, slot_b, res_v, bnd_s, rsa, rsb, osem):
    wid = lax.axis_index("s") * _NC + lax.axis_index("c")
    dt = wid // 8
    r = wid % 8
    slots = (slot_a, slot_b)
    rs = (rsa, rsb)

    pltpu.sync_copy(bnd_hbm, bnd_s)
    pltpu.sync_copy(pk_hbm, pk_v)
    bv = bnd_s[pl.ds(0, 16)]
    glo = [bv[c] for c in range(_NCH)]
    ghi = [bv[_NCH + c] for c in range(_NCH)]

    pltpu.async_copy(tab_hbm.at[0, wid, pl.ds(0, _CH)], slot_a, rsa)

    def unit(t, carry):
        pltpu.make_async_copy(
            tab_hbm.at[t, wid, pl.ds(0, _CH)], slot_a, rsa).wait()
        prev = None
        prev_tail = None
        for c in range(_NCH):
            s = c % 2
            tail_cp = None
            if c + 1 < _NCH:
                nxt = pltpu.async_copy(
                    tab_hbm.at[t, wid, pl.ds((c + 1) * _CH, _SZ[c + 1])],
                    slots[(c + 1) % 2].at[pl.ds(0, _SZ[c + 1])],
                    rs[(c + 1) % 2])
                if c + 1 == _NCH - 1:
                    tail_cp = pltpu.async_copy(
                        tail_hbm.at[t, wid],
                        slots[(c + 1) % 2].at[pl.ds(_CHB, 128)],
                        rs[(c + 1) % 2])
            else:
                tn = lax.rem(t + 1, _N_TABLES)
                nxt = pltpu.async_copy(
                    tab_hbm.at[tn, wid, pl.ds(0, _CH)], slot_a, rsa)
            if prev is not None:
                prev.wait()
            if prev_tail is not None:
                prev_tail.wait()
            slot = slots[s]

            def sel(g, carry2):
                pk = pk_v[pl.ds(g * 16, 16)]
                xv = pk & 16383
                pv = lax.shift_right_logical(pk, 14) & 16383
                cid = lax.shift_right_logical(pk, 28)
                vals = plsc.load_gather(slot, [xv])
                plsc.store_scatter(res_v, [pv >> 7, pv & 127], vals,
                                   mask=cid == c)
                return carry2

            lax.fori_loop(glo[c], ghi[c], sel, 0)
            prev = nxt
            prev_tail = tail_cp
        ocs = [pltpu.async_copy(
            res_v.at[pl.ds(h * 32, 32)],
            out_hbm.at[t, dt, pl.ds(h * 32, 32), r], osem) for h in range(4)]
        for o in ocs:
            o.wait()
        return carry

    lax.fori_loop(0, _N_TABLES, unit, 0)
    pltpu.make_async_copy(
        tab_hbm.at[0, wid, pl.ds(0, _CH)], slot_a, rsa).wait()


def kernel(x, tables):
    tab_t = jnp.transpose(tables, (0, 2, 1))
    tail = jnp.zeros((_N_TABLES, _D, 128), jnp.float32).at[:, :, :_TAIL].set(
        jnp.transpose(tables[:, 7 * _CH + _CHB:, :], (0, 2, 1)))
    xi = x.astype(jnp.int32)
    xs, order = lax.sort_key_val(xi, jnp.arange(_B, dtype=jnp.int32))
    cid = xs // _CH
    local = xs - cid * _CH
    pk = local | (order << 14) | (cid << 28)
    cnt = jnp.sum((cid[:, None] == jnp.arange(_NCH, dtype=jnp.int32)[None, :])
                  .astype(jnp.int32), axis=0)
    cum = jnp.concatenate(
        [jnp.zeros((1,), jnp.int32), jnp.cumsum(cnt, dtype=jnp.int32)])
    glo = cum[:_NCH] // 16
    ghi = (cum[1:] + 15) // 16
    bnd = jnp.concatenate([glo, ghi]).astype(jnp.int32)

    run = pl.kernel(
        _body,
        out_type=jax.ShapeDtypeStruct((_N_TABLES, 4, _B // 128, 8, 128),
                                      jnp.float32),
        mesh=plsc.VectorSubcoreMesh(
            core_axis_name="c", subcore_axis_name="s",
            num_cores=_NC, num_subcores=_NS),
        scratch_types=[
            pltpu.VMEM((_B,), jnp.int32),
            pltpu.VMEM((_CH,), jnp.float32),
            pltpu.VMEM((_CH,), jnp.float32),
            pltpu.VMEM((_B // 128, 128), jnp.float32),
            pltpu.VMEM((2 * _NCH,), jnp.int32),
            pltpu.SemaphoreType.DMA,
            pltpu.SemaphoreType.DMA,
            pltpu.SemaphoreType.DMA,
        ],
        compiler_params=pltpu.CompilerParams(
            use_tc_tiling_on_sc=True, needs_layout_passes=False),
    )
    out5d = run(pk, bnd, tab_t, tail)
    out = out5d.transpose(2, 4, 0, 1, 3).reshape(_B, _N_TABLES, _D)
    return out


# speedup vs baseline: 1.0065x; 1.0065x over previous
import functools

import jax
import jax.numpy as jnp
from jax import lax
from jax.experimental import pallas as pl
from jax.experimental.pallas import tpu as pltpu
from jax.experimental.pallas import tpu_sc as plsc

_N_TABLES = 26
_VOCAB = 100000
_D = 32
_B = 16384
_NC = 2
_NS = 16
_NW = _NC * _NS
_NCH = 4
_CH = 25088
_CHB = 24704
_TAIL = _VOCAB - 3 * _CH - _CHB
_SZ = [_CH] * 3 + [_CHB]


def _body(pk_hbm, bnd_hbm, tab_hbm, tail_hbm, out_hbm,
          pk_v, slot_a, slot_b, res_v, bnd_s, rsa, rsb, osem):
    wid = lax.axis_index("s") * _NC + lax.axis_index("c")
    dt = wid // 8
    r = wid % 8
    slots = (slot_a, slot_b)
    rs = (rsa, rsb)

    pltpu.sync_copy(bnd_hbm, bnd_s)
    pltpu.sync_copy(pk_hbm, pk_v)
    bv = bnd_s[pl.ds(0, 16)]
    glo = [bv[c] for c in range(_NCH)]
    ghi = [bv[_NCH + c] for c in range(_NCH)]

    pltpu.async_copy(tab_hbm.at[0, wid, pl.ds(0, _CH)], slot_a, rsa)

    def unit(t, carry):
        pltpu.make_async_copy(
            tab_hbm.at[t, wid, pl.ds(0, _CH)], slot_a, rsa).wait()
        prev = None
        prev_tail = None
        for c in range(_NCH):
            s = c % 2
            tail_cp = None
            if c + 1 < _NCH:
                nxt = pltpu.async_copy(
                    tab_hbm.at[t, wid, pl.ds((c + 1) * _CH, _SZ[c + 1])],
                    slots[(c + 1) % 2].at[pl.ds(0, _SZ[c + 1])],
                    rs[(c + 1) % 2])
                if c + 1 == _NCH - 1:
                    tail_cp = pltpu.async_copy(
                        tail_hbm.at[t, wid],
                        slots[(c + 1) % 2].at[pl.ds(_CHB, 128)],
                        rs[(c + 1) % 2])
            else:
                tn = lax.rem(t + 1, _N_TABLES)
                nxt = pltpu.async_copy(
                    tab_hbm.at[tn, wid, pl.ds(0, _CH)], slot_a, rsa)
            if prev is not None:
                prev.wait()
            if prev_tail is not None:
                prev_tail.wait()
            slot = slots[s]

            def sel(g, carry2):
                pk = pk_v[pl.ds(g * 16, 16)]
                xv = pk & 32767
                pv = lax.shift_right_logical(pk, 15) & 16383
                cid = lax.shift_right_logical(pk, 29)
                vals = plsc.load_gather(slot, [xv])
                plsc.store_scatter(res_v, [pv >> 7, pv & 127], vals,
                                   mask=cid == c)
                return carry2

            lax.fori_loop(glo[c], ghi[c], sel, 0)
            prev = nxt
            prev_tail = tail_cp
        ocs = [pltpu.async_copy(
            res_v.at[pl.ds(h * 32, 32)],
            out_hbm.at[t, dt, pl.ds(h * 32, 32), r], osem) for h in range(4)]
        for o in ocs:
            o.wait()
        return carry

    lax.fori_loop(0, _N_TABLES, unit, 0)
    pltpu.make_async_copy(
        tab_hbm.at[0, wid, pl.ds(0, _CH)], slot_a, rsa).wait()


def kernel(x, tables):
    tab_t = jnp.transpose(tables, (0, 2, 1))
    tail = jnp.zeros((_N_TABLES, _D, 128), jnp.float32).at[:, :, :_TAIL].set(
        jnp.transpose(tables[:, 3 * _CH + _CHB:, :], (0, 2, 1)))
    xi = x.astype(jnp.int32)
    xs, order = lax.sort_key_val(xi, jnp.arange(_B, dtype=jnp.int32))
    cid = xs // _CH
    local = xs - cid * _CH
    pk = local | (order << 15) | (cid << 29)
    cnt = jnp.sum((cid[:, None] == jnp.arange(_NCH, dtype=jnp.int32)[None, :])
                  .astype(jnp.int32), axis=0)
    cum = jnp.concatenate(
        [jnp.zeros((1,), jnp.int32), jnp.cumsum(cnt, dtype=jnp.int32)])
    glo = cum[:_NCH] // 16
    ghi = (cum[1:] + 15) // 16
    bnd = jnp.concatenate(
        [glo, ghi, jnp.zeros((16 - 2 * _NCH,), jnp.int32)]).astype(jnp.int32)

    run = pl.kernel(
        _body,
        out_type=jax.ShapeDtypeStruct((_N_TABLES, 4, _B // 128, 8, 128),
                                      jnp.float32),
        mesh=plsc.VectorSubcoreMesh(
            core_axis_name="c", subcore_axis_name="s",
            num_cores=_NC, num_subcores=_NS),
        scratch_types=[
            pltpu.VMEM((_B,), jnp.int32),
            pltpu.VMEM((_CH,), jnp.float32),
            pltpu.VMEM((_CH,), jnp.float32),
            pltpu.VMEM((_B // 128, 128), jnp.float32),
            pltpu.VMEM((16,), jnp.int32),
            pltpu.SemaphoreType.DMA,
            pltpu.SemaphoreType.DMA,
            pltpu.SemaphoreType.DMA,
        ],
        compiler_params=pltpu.CompilerParams(
            use_tc_tiling_on_sc=True, needs_layout_passes=False),
    )
    out5d = run(pk, bnd, tab_t, tail)
    out = out5d.transpose(2, 4, 0, 1, 3).reshape(_B, _N_TABLES, _D)
    return out


# speedup vs baseline: 1.0139x; 1.0073x over previous
import functools

import jax
import jax.numpy as jnp
from jax import lax
from jax.experimental import pallas as pl
from jax.experimental.pallas import tpu as pltpu
from jax.experimental.pallas import tpu_sc as plsc

_N_TABLES = 26
_VOCAB = 100000
_D = 32
_B = 16384
_NC = 2
_NS = 16
_NW = _NC * _NS
_NCH = 4
_CH = 25088
_CHB = 24704
_TAIL = _VOCAB - 3 * _CH - _CHB
_SZ = [_CH] * 3 + [_CHB]


def _body(pk_hbm, bnd_hbm, tab_hbm, tail_hbm, out_hbm,
          pk_v, slot_a, slot_b, res_v, bnd_s, rsa, rsb, osem):
    wid = lax.axis_index("s") * _NC + lax.axis_index("c")
    dt = wid // 8
    r = wid % 8
    slots = (slot_a, slot_b)
    rs = (rsa, rsb)

    pltpu.sync_copy(bnd_hbm, bnd_s)
    pltpu.sync_copy(pk_hbm, pk_v)
    bv = bnd_s[pl.ds(0, 16)]
    glo = [bv[c] for c in range(_NCH)]
    ghi = [bv[_NCH + c] for c in range(_NCH)]

    pltpu.async_copy(tab_hbm.at[0, wid, pl.ds(0, _CH)], slot_a, rsa)

    def unit(t, carry):
        pltpu.make_async_copy(
            tab_hbm.at[t, wid, pl.ds(0, _CH)], slot_a, rsa).wait()
        prev = None
        prev_tail = None
        for c in range(_NCH):
            s = c % 2
            tail_cp = None
            if c + 1 < _NCH:
                nxt = pltpu.async_copy(
                    tab_hbm.at[t, wid, pl.ds((c + 1) * _CH, _SZ[c + 1])],
                    slots[(c + 1) % 2].at[pl.ds(0, _SZ[c + 1])],
                    rs[(c + 1) % 2])
                if c + 1 == _NCH - 1:
                    tail_cp = pltpu.async_copy(
                        tail_hbm.at[t, wid],
                        slots[(c + 1) % 2].at[pl.ds(_CHB, 128)],
                        rs[(c + 1) % 2])
            else:
                tn = lax.rem(t + 1, _N_TABLES)
                nxt = pltpu.async_copy(
                    tab_hbm.at[tn, wid, pl.ds(0, _CH)], slot_a, rsa)
            if prev is not None:
                prev.wait()
            if prev_tail is not None:
                prev_tail.wait()
            slot = slots[s]

            def sel(gg, carry2):
                for u in range(8):
                    pk = pk_v[pl.ds(gg * 128 + u * 16, 16)]
                    xv = pk & 32767
                    pv = lax.shift_right_logical(pk, 15) & 16383
                    cid = lax.shift_right_logical(pk, 29)
                    vals = plsc.load_gather(slot, [xv])
                    plsc.store_scatter(res_v, [pv >> 7, pv & 127], vals,
                                       mask=cid == c)
                return carry2

            lax.fori_loop(glo[c] >> 3, (ghi[c] + 7) >> 3, sel, 0)
            prev = nxt
            prev_tail = tail_cp
        ocs = [pltpu.async_copy(
            res_v.at[pl.ds(h * 32, 32)],
            out_hbm.at[t, dt, pl.ds(h * 32, 32), r], osem) for h in range(4)]
        for o in ocs:
            o.wait()
        return carry

    lax.fori_loop(0, _N_TABLES, unit, 0)
    pltpu.make_async_copy(
        tab_hbm.at[0, wid, pl.ds(0, _CH)], slot_a, rsa).wait()


def kernel(x, tables):
    tab_t = jnp.transpose(tables, (0, 2, 1))
    tail = jnp.zeros((_N_TABLES, _D, 128), jnp.float32).at[:, :, :_TAIL].set(
        jnp.transpose(tables[:, 3 * _CH + _CHB:, :], (0, 2, 1)))
    xi = x.astype(jnp.int32)
    xs, order = lax.sort_key_val(xi, jnp.arange(_B, dtype=jnp.int32))
    cid = xs // _CH
    local = xs - cid * _CH
    pk = local | (order << 15) | (cid << 29)
    cnt = jnp.sum((cid[:, None] == jnp.arange(_NCH, dtype=jnp.int32)[None, :])
                  .astype(jnp.int32), axis=0)
    cum = jnp.concatenate(
        [jnp.zeros((1,), jnp.int32), jnp.cumsum(cnt, dtype=jnp.int32)])
    glo = cum[:_NCH] // 16
    ghi = (cum[1:] + 15) // 16
    bnd = jnp.concatenate(
        [glo, ghi, jnp.zeros((16 - 2 * _NCH,), jnp.int32)]).astype(jnp.int32)

    run = pl.kernel(
        _body,
        out_type=jax.ShapeDtypeStruct((_N_TABLES, 4, _B // 128, 8, 128),
                                      jnp.float32),
        mesh=plsc.VectorSubcoreMesh(
            core_axis_name="c", subcore_axis_name="s",
            num_cores=_NC, num_subcores=_NS),
        scratch_types=[
            pltpu.VMEM((_B,), jnp.int32),
            pltpu.VMEM((_CH,), jnp.float32),
            pltpu.VMEM((_CH,), jnp.float32),
            pltpu.VMEM((_B // 128, 128), jnp.float32),
            pltpu.VMEM((16,), jnp.int32),
            pltpu.SemaphoreType.DMA,
            pltpu.SemaphoreType.DMA,
            pltpu.SemaphoreType.DMA,
        ],
        compiler_params=pltpu.CompilerParams(
            use_tc_tiling_on_sc=True, needs_layout_passes=False),
    )
    out5d = run(pk, bnd, tab_t, tail)
    out = out5d.transpose(2, 4, 0, 1, 3).reshape(_B, _N_TABLES, _D)
    return out


# speedup vs baseline: 1.2349x; 1.2180x over previous
import functools

import jax
import jax.numpy as jnp
from jax import lax
from jax.experimental import pallas as pl
from jax.experimental.pallas import tpu as pltpu
from jax.experimental.pallas import tpu_sc as plsc

_N_TABLES = 26
_VOCAB = 100000
_D = 32
_B = 16384
_NC = 2
_NS = 16
_NW = _NC * _NS
_Q = _B // 4
_QR = _Q // 128


def _body(x_hbm, tab_hbm, out_hbm, row_v, x_v, res_a, res_b, osem):
    wid = lax.axis_index("s") * _NC + lax.axis_index("c")
    dt = wid // 8
    r = wid % 8
    res = (res_a, res_b)
    pending = [None, None]

    pltpu.sync_copy(x_hbm, x_v)
    step = 0
    for t in range(_N_TABLES):
        pltpu.sync_copy(tab_hbm.at[t, wid], row_v)
        for h in range(4):
            slot = step % 2
            buf = res[slot]
            if pending[slot] is not None:
                pending[slot].wait()

            def sel(row, carry):
                for c in range(8):
                    xv = x_v[pl.ds(h * _Q + row * 128 + c * 16, 16)]
                    buf[row, pl.ds(c * 16, 16)] = plsc.load_gather(
                        row_v, [xv])
                return carry

            lax.fori_loop(0, _QR, sel, 0)

            pending[slot] = pltpu.async_copy(
                buf, out_hbm.at[t, dt, pl.ds(h * _QR, _QR), r], osem)
            step += 1
    for cp in pending:
        if cp is not None:
            cp.wait()


def kernel(x, tables):
    tab_t = jnp.transpose(tables, (0, 2, 1))
    run = pl.kernel(
        _body,
        out_type=jax.ShapeDtypeStruct((_N_TABLES, 4, _B // 128, 8, 128),
                                      jnp.float32),
        mesh=plsc.VectorSubcoreMesh(
            core_axis_name="c", subcore_axis_name="s",
            num_cores=_NC, num_subcores=_NS),
        scratch_types=[
            pltpu.VMEM((_VOCAB,), jnp.float32),
            pltpu.VMEM((_B,), jnp.int32),
            pltpu.VMEM((_QR, 128), jnp.float32),
            pltpu.VMEM((_QR, 128), jnp.float32),
            pltpu.SemaphoreType.DMA,
        ],
        compiler_params=pltpu.CompilerParams(
            use_tc_tiling_on_sc=True, needs_layout_passes=False),
    )
    out5d = run(x.astype(jnp.int32), tab_t)
    out = out5d.transpose(2, 4, 0, 1, 3).reshape(_B, _N_TABLES, _D)
    return out
